# Initial kernel scaffold; baseline (speedup 1.0000x reference)
import jax
import jax.numpy as jnp
from jax.experimental import pallas as pl


def kernel(x, adj, W1, b1, W2, b2):
    raise NotImplementedError("write your pallas kernel here")


# speedup vs baseline: 29.1191x; 29.1191x over previous
import functools

import jax
import jax.numpy as jnp
from jax import lax
from jax.experimental import pallas as pl

N = 1024
C = 128


def _gcn_body(x_ref, adj_ref, w1_ref, b1_ref, w2_ref, b2_ref, out_ref):
    adj = adj_ref[0]
    x = x_ref[0]

    n_elem = N * N
    k1 = 734002
    c_target = float(n_elem - 1 - k1)

    def bisect(i, carry):
        lo, hi, c_hi = carry
        mid = 0.5 * (lo + hi)
        c = jnp.sum(jnp.where(adj > mid, 1.0, 0.0))
        gt = c > c_target
        return (jnp.where(gt, mid, lo),
                jnp.where(gt, hi, mid),
                jnp.where(gt, c_hi, c))

    lo, hi, c_hi = lax.fori_loop(
        0, 32, bisect,
        (jnp.float32(-1.0), jnp.float32(1.0), jnp.float32(0.0)))

    v1 = jnp.max(jnp.where(adj <= hi, adj, -1.0))
    v2_cand = jnp.min(jnp.where(adj > hi, adj, 2.0))
    v2 = jnp.where(c_hi <= c_target - 1.0, v1, v2_cand)
    thresh = 0.5 * v1 + 0.5 * v2

    mask = (adj > thresh).astype(jnp.float32)
    xw = jnp.dot(x, w1_ref[:], preferred_element_type=jnp.float32) + b1_ref[:]
    h = jnp.maximum(jnp.dot(mask, xw, preferred_element_type=jnp.float32), 0.0)
    hw = jnp.dot(h, w2_ref[:], preferred_element_type=jnp.float32) + b2_ref[:]
    h2 = jnp.maximum(jnp.dot(mask, hw, preferred_element_type=jnp.float32), 0.0)
    out_ref[:] = (jnp.sum(h2, axis=0) * (1.0 / N)).reshape(1, 1, C)


def kernel(x, adj, W1, b1, W2, b2):
    b = adj.shape[0]
    out = pl.pallas_call(
        _gcn_body,
        grid=(b,),
        in_specs=[
            pl.BlockSpec((1, N, C), lambda i: (i, 0, 0)),
            pl.BlockSpec((1, N, N), lambda i: (i, 0, 0)),
            pl.BlockSpec((C, C), lambda i: (0, 0)),
            pl.BlockSpec((1, C), lambda i: (0, 0)),
            pl.BlockSpec((C, C), lambda i: (0, 0)),
            pl.BlockSpec((1, C), lambda i: (0, 0)),
        ],
        out_specs=pl.BlockSpec((1, 1, C), lambda i: (i, 0, 0)),
        out_shape=jax.ShapeDtypeStruct((b, 1, C), jnp.float32),
    )(x, adj, W1, b1.reshape(1, C), W2, b2.reshape(1, C))
    return out.reshape(b, C)
